# baseline (speedup 1.0000x reference)
import functools

import jax
import jax.numpy as jnp
from jax import lax
from jax.experimental import pallas as pl
from jax.experimental.pallas import tpu as pltpu
from jax.experimental.pallas import tpu_sc as plsc

NUM_EMB = 1000000
DIM = 64
B = 16384
L = 50
MAX_NORM = 3.0

NC = 2
NS = 16
NW = NC * NS
R = B * L
IW = 128
CH = 512
ROWS_PER_W = R // NW
CHUNKS = ROWS_PER_W // CH
STREAMS = CH // IW
GROUPS = CH // 16


def _rescale_group(rows_v, rid):
    accs = [jnp.zeros((16,), jnp.float32) for _ in range(4)]
    for c in range(DIM):
        col = jnp.full((16,), c, jnp.int32)
        v = plsc.load_gather(rows_v, [rid, col])
        accs[c % 4] = accs[c % 4] + v * v
    s = (accs[0] + accs[1]) + (accs[2] + accs[3])
    ii = plsc.bitcast(s, jnp.int32)
    ii = 0x5F3759DF - (ii >> 1)
    y = plsc.bitcast(ii, jnp.float32)
    for _ in range(3):
        y = y * (1.5 - 0.5 * s * y * y)
    scale = jnp.minimum(jnp.float32(1.0), jnp.float32(MAX_NORM) * y)
    for c in range(DIM):
        col = jnp.full((16,), c, jnp.int32)
        v = plsc.load_gather(rows_v, [rid, col])
        plsc.store_scatter(rows_v, [rid, col], v * scale)


@functools.partial(
    pl.kernel,
    out_type=jax.ShapeDtypeStruct((R, DIM), jnp.float32),
    mesh=plsc.VectorSubcoreMesh(core_axis_name="c", subcore_axis_name="s"),
    compiler_params=pltpu.CompilerParams(
        needs_layout_passes=False, use_tc_tiling_on_sc=False
    ),
    scratch_types=[
        pltpu.VMEM((STREAMS, IW), jnp.int32),
        pltpu.VMEM((CH, DIM), jnp.float32),
        pltpu.SemaphoreType.DMA,
    ],
)
def _sc_lookup(idx_hbm, table_hbm, out_hbm, idx_v, rows_v, sem):
    wid = lax.axis_index("s") * NC + lax.axis_index("c")
    iota16 = lax.iota(jnp.int32, 16)

    def chunk_body(t, carry):
        base = wid * ROWS_PER_W + t * CH
        r0 = wid * (ROWS_PER_W // IW) + t * STREAMS
        pltpu.sync_copy(idx_hbm.at[pl.ds(r0, STREAMS)], idx_v)
        handles = [
            pltpu.async_copy(
                table_hbm.at[idx_v.at[j]],
                rows_v.at[pl.ds(j * IW, IW)],
                sem,
            )
            for j in range(STREAMS)
        ]
        for h in handles:
            h.wait()

        def group_body(g, c2):
            _rescale_group(rows_v, g * 16 + iota16)
            return c2

        lax.fori_loop(0, GROUPS, group_body, 0)
        pltpu.sync_copy(rows_v, out_hbm.at[pl.ds(base, CH)])
        return carry

    lax.fori_loop(0, CHUNKS, chunk_body, 0)


def kernel(nouns_idx_tensor, vis_embeddings):
    idx2 = nouns_idx_tensor.reshape(R // IW, IW)
    out = _sc_lookup(idx2, vis_embeddings)
    return out.reshape(B, L, DIM)


# speedup vs baseline: 1.1464x; 1.1464x over previous
import functools

import jax
import jax.numpy as jnp
from jax import lax
from jax.experimental import pallas as pl
from jax.experimental.pallas import tpu as pltpu
from jax.experimental.pallas import tpu_sc as plsc

NUM_EMB = 1000000
DIM = 64
B = 16384
L = 50
MAX_NORM = 3.0

NC = 2
NS = 16
NW = NC * NS
R = B * L
IW = 128
CH = 512
ROWS_PER_W = R // NW
CHUNKS = ROWS_PER_W // CH
STREAMS = CH // IW
GROUPS = CH // 16


def _rescale_group(rows_v, rid):
    accs = [jnp.zeros((16,), jnp.float32) for _ in range(4)]
    for c in range(DIM):
        col = jnp.full((16,), c, jnp.int32)
        v = plsc.load_gather(rows_v, [rid, col])
        accs[c % 4] = accs[c % 4] + v * v
    s = (accs[0] + accs[1]) + (accs[2] + accs[3])
    ii = plsc.bitcast(s, jnp.int32)
    ii = 0x5F3759DF - (ii >> 1)
    y = plsc.bitcast(ii, jnp.float32)
    for _ in range(3):
        y = y * (1.5 - 0.5 * s * y * y)
    scale = jnp.minimum(jnp.float32(1.0), jnp.float32(MAX_NORM) * y)
    for c in range(DIM):
        col = jnp.full((16,), c, jnp.int32)
        v = plsc.load_gather(rows_v, [rid, col])
        plsc.store_scatter(rows_v, [rid, col], v * scale)


@functools.partial(
    pl.kernel,
    out_type=jax.ShapeDtypeStruct((R, DIM), jnp.float32),
    mesh=plsc.VectorSubcoreMesh(core_axis_name="c", subcore_axis_name="s"),
    compiler_params=pltpu.CompilerParams(
        needs_layout_passes=False, use_tc_tiling_on_sc=False
    ),
    scratch_types=[
        pltpu.VMEM((STREAMS, IW), jnp.int32),
        pltpu.VMEM((CH, DIM), jnp.float32),
        pltpu.SemaphoreType.DMA,
    ],
)
def _sc_lookup(idx_hbm, table_hbm, out_hbm, idx_v, rows_v, sem):
    wid = lax.axis_index("s") * NC + lax.axis_index("c")
    iota16 = lax.iota(jnp.int32, 16)

    def chunk_body(t, carry):
        base = wid * ROWS_PER_W + t * CH
        r0 = wid * (ROWS_PER_W // IW) + t * STREAMS
        pltpu.sync_copy(idx_hbm.at[pl.ds(r0, STREAMS)], idx_v)
        handles = [
            pltpu.async_copy(
                table_hbm.at[idx_v.at[j]],
                rows_v.at[pl.ds(j * IW, IW)],
                sem,
            )
            for j in range(STREAMS)
        ]
        for h in handles:
            h.wait()

        @plsc.parallel_loop(0, GROUPS, unroll=2)
        def _(g):
            _rescale_group(rows_v, g * 16 + iota16)
        pltpu.sync_copy(rows_v, out_hbm.at[pl.ds(base, CH)])
        return carry

    lax.fori_loop(0, CHUNKS, chunk_body, 0)


def kernel(nouns_idx_tensor, vis_embeddings):
    idx2 = nouns_idx_tensor.reshape(R // IW, IW)
    out = _sc_lookup(idx2, vis_embeddings)
    return out.reshape(B, L, DIM)


# speedup vs baseline: 2.4832x; 2.1661x over previous
import functools

import jax
import jax.numpy as jnp
import numpy as np
from jax import lax
from jax.experimental import pallas as pl
from jax.experimental.pallas import tpu as pltpu
from jax.experimental.pallas import tpu_sc as plsc

NUM_EMB = 1000000
DIM = 64
B = 16384
L = 50
MAX_NORM = 3.0

NC = 2
NS = 16
NW = NC * NS
R = B * L
IW = 128
CH = 512
ROWS_PER_W = R // NW
CHUNKS = ROWS_PER_W // CH
STREAMS = CH // IW
GROUPS = CH // 16


def _rescale_row(rows_v, r, rot_idx):
    a = [rows_v[r, pl.ds(q * 16, 16)] for q in range(DIM // 16)]
    sq = a[0] * a[0]
    for q in range(1, DIM // 16):
        sq = sq + a[q] * a[q]
    for rot in rot_idx:
        sq = sq + jnp.take_along_axis(sq, rot, axis=0)
    ii = plsc.bitcast(sq, jnp.int32)
    ii = 0x5F3759DF - (ii >> 1)
    y = plsc.bitcast(ii, jnp.float32)
    for _ in range(3):
        y = y * (1.5 - 0.5 * sq * y * y)
    scale = jnp.minimum(jnp.float32(1.0), jnp.float32(MAX_NORM) * y)
    for q in range(DIM // 16):
        rows_v[r, pl.ds(q * 16, 16)] = a[q] * scale


@functools.partial(
    pl.kernel,
    out_type=jax.ShapeDtypeStruct((R, DIM), jnp.float32),
    mesh=plsc.VectorSubcoreMesh(core_axis_name="c", subcore_axis_name="s"),
    compiler_params=pltpu.CompilerParams(
        needs_layout_passes=False, use_tc_tiling_on_sc=False
    ),
    scratch_types=[
        pltpu.VMEM((STREAMS, IW), jnp.int32),
        pltpu.VMEM((CH, DIM), jnp.float32),
        pltpu.SemaphoreType.DMA,
    ],
)
def _sc_lookup(idx_hbm, table_hbm, out_hbm, idx_v, rows_v, sem):
    wid = lax.axis_index("s") * NC + lax.axis_index("c")
    iota16 = lax.iota(jnp.int32, 16)
    rot_idx = [iota16 ^ (1 << j) for j in range(4)]

    def chunk_body(t, carry):
        base = wid * ROWS_PER_W + t * CH
        r0 = wid * (ROWS_PER_W // IW) + t * STREAMS
        pltpu.sync_copy(idx_hbm.at[pl.ds(r0, STREAMS)], idx_v)
        handles = [
            pltpu.async_copy(
                table_hbm.at[idx_v.at[j]],
                rows_v.at[pl.ds(j * IW, IW)],
                sem,
            )
            for j in range(STREAMS)
        ]
        for h in handles:
            h.wait()

        @plsc.parallel_loop(0, CH, unroll=4)
        def _(r):
            _rescale_row(rows_v, r, rot_idx)
        pltpu.sync_copy(rows_v, out_hbm.at[pl.ds(base, CH)])
        return carry

    lax.fori_loop(0, CHUNKS, chunk_body, 0)


def kernel(nouns_idx_tensor, vis_embeddings):
    idx2 = nouns_idx_tensor.reshape(R // IW, IW)
    out = _sc_lookup(idx2, vis_embeddings)
    return out.reshape(B, L, DIM)


# speedup vs baseline: 2.6135x; 1.0525x over previous
import functools

import jax
import jax.numpy as jnp
import numpy as np
from jax import lax
from jax.experimental import pallas as pl
from jax.experimental.pallas import tpu as pltpu
from jax.experimental.pallas import tpu_sc as plsc

NUM_EMB = 1000000
DIM = 64
B = 16384
L = 50
MAX_NORM = 3.0

NC = 2
NS = 16
NW = NC * NS
R = B * L
IW = 128
CH = 512
ROWS_PER_W = R // NW
CHUNKS = ROWS_PER_W // CH
STREAMS = CH // IW
GROUPS = CH // 16


def _rescale_row(rows_v, out_v, r, half, rot_idx):
    a = [rows_v[2 * r + half, pl.ds(q * 16, 16)] for q in range(DIM // 16)]
    sq = a[0] * a[0]
    for q in range(1, DIM // 16):
        sq = sq + a[q] * a[q]
    for rot in rot_idx:
        sq = sq + jnp.take_along_axis(sq, rot, axis=0)
    ii = plsc.bitcast(sq, jnp.int32)
    ii = 0x5F3759DF - (ii >> 1)
    y = plsc.bitcast(ii, jnp.float32)
    for _ in range(3):
        y = y * (1.5 - 0.5 * sq * y * y)
    scale = jnp.minimum(jnp.float32(1.0), jnp.float32(MAX_NORM) * y)
    for q in range(DIM // 16):
        out_v[r, pl.ds(half * DIM + q * 16, 16)] = a[q] * scale


@functools.partial(
    pl.kernel,
    out_type=jax.ShapeDtypeStruct((R // 2, 2 * DIM), jnp.float32),
    mesh=plsc.VectorSubcoreMesh(core_axis_name="c", subcore_axis_name="s"),
    compiler_params=pltpu.CompilerParams(
        needs_layout_passes=False, use_tc_tiling_on_sc=False
    ),
    scratch_types=[
        pltpu.VMEM((STREAMS, IW), jnp.int32),
        pltpu.VMEM((CH, DIM), jnp.float32),
        pltpu.VMEM((CH // 2, 2 * DIM), jnp.float32),
        pltpu.SemaphoreType.DMA,
    ],
)
def _sc_lookup(idx_hbm, table_hbm, out_hbm, idx_v, rows_v, out_v, sem):
    wid = lax.axis_index("s") * NC + lax.axis_index("c")
    iota16 = lax.iota(jnp.int32, 16)
    rot_idx = [iota16 ^ (1 << j) for j in range(4)]

    def chunk_body(t, carry):
        base = wid * ROWS_PER_W + t * CH
        r0 = wid * (ROWS_PER_W // IW) + t * STREAMS
        pltpu.sync_copy(idx_hbm.at[pl.ds(r0, STREAMS)], idx_v)
        handles = [
            pltpu.async_copy(
                table_hbm.at[idx_v.at[j]],
                rows_v.at[pl.ds(j * IW, IW)],
                sem,
            )
            for j in range(STREAMS)
        ]
        for h in handles:
            h.wait()

        @plsc.parallel_loop(0, CH // 2, unroll=2)
        def _(r):
            _rescale_row(rows_v, out_v, r, 0, rot_idx)
            _rescale_row(rows_v, out_v, r, 1, rot_idx)

        pltpu.sync_copy(out_v, out_hbm.at[pl.ds(base // 2, CH // 2)])
        return carry

    lax.fori_loop(0, CHUNKS, chunk_body, 0)


def kernel(nouns_idx_tensor, vis_embeddings):
    idx2 = nouns_idx_tensor.reshape(R // IW, IW)
    out = _sc_lookup(idx2, vis_embeddings)
    return out.reshape(B, L, DIM)


# speedup vs baseline: 2.8242x; 1.0806x over previous
import functools

import jax
import jax.numpy as jnp
import numpy as np
from jax import lax
from jax.experimental import pallas as pl
from jax.experimental.pallas import tpu as pltpu
from jax.experimental.pallas import tpu_sc as plsc

NUM_EMB = 1000000
DIM = 64
B = 16384
L = 50
MAX_NORM = 3.0

NC = 2
NS = 16
NW = NC * NS
R = B * L
IW = 128
CH = 256
ROWS_PER_W = R // NW
CHUNKS = ROWS_PER_W // CH
STREAMS = CH // IW


def _rescale_row(rows_v, out_v, r, half, rot_idx):
    a = [rows_v[2 * r + half, pl.ds(q * 16, 16)] for q in range(DIM // 16)]
    sq = a[0] * a[0]
    for q in range(1, DIM // 16):
        sq = sq + a[q] * a[q]
    for rot in rot_idx:
        sq = sq + jnp.take_along_axis(sq, rot, axis=0)
    ii = plsc.bitcast(sq, jnp.int32)
    ii = 0x5F3759DF - (ii >> 1)
    y = plsc.bitcast(ii, jnp.float32)
    for _ in range(3):
        y = y * (1.5 - 0.5 * sq * y * y)
    scale = jnp.minimum(jnp.float32(1.0), jnp.float32(MAX_NORM) * y)
    for q in range(DIM // 16):
        out_v[r, pl.ds(half * DIM + q * 16, 16)] = a[q] * scale


@functools.partial(
    pl.kernel,
    out_type=jax.ShapeDtypeStruct((R // 2, 2 * DIM), jnp.float32),
    mesh=plsc.VectorSubcoreMesh(core_axis_name="c", subcore_axis_name="s"),
    compiler_params=pltpu.CompilerParams(
        needs_layout_passes=False, use_tc_tiling_on_sc=False
    ),
    scratch_types=[
        pltpu.VMEM((2, STREAMS, IW), jnp.int32),
        pltpu.VMEM((2, CH, DIM), jnp.float32),
        pltpu.VMEM((2, CH // 2, 2 * DIM), jnp.float32),
        pltpu.SemaphoreType.DMA,
        pltpu.SemaphoreType.DMA,
        pltpu.SemaphoreType.DMA,
        pltpu.SemaphoreType.DMA,
    ],
)
def _sc_lookup(
    idx_hbm, table_hbm, out_hbm, idx_v, rows_v, out_v, g0, g1, o0, o1
):
    wid = lax.axis_index("s") * NC + lax.axis_index("c")
    iota16 = lax.iota(jnp.int32, 16)
    rot_idx = [iota16 ^ (1 << j) for j in range(4)]
    gsem = (g0, g1)
    osem = (o0, o1)

    def gather_descs(t, b):
        return [
            pltpu.make_async_copy(
                table_hbm.at[idx_v.at[b, j]],
                rows_v.at[b, pl.ds(j * IW, IW)],
                gsem[b],
            )
            for j in range(STREAMS)
        ]

    def prefetch(t, b):
        r0 = wid * (ROWS_PER_W // IW) + t * STREAMS
        pltpu.sync_copy(idx_hbm.at[pl.ds(r0, STREAMS)], idx_v.at[b])
        for d in gather_descs(t, b):
            d.start()

    def out_desc(t, b):
        base2 = (wid * ROWS_PER_W + t * CH) // 2
        return pltpu.make_async_copy(
            out_v.at[b], out_hbm.at[pl.ds(base2, CH // 2)], osem[b]
        )

    def process(t, b, drain_out):
        for d in gather_descs(t, b):
            d.wait()
        if drain_out:
            out_desc(t, b).wait()

        @plsc.parallel_loop(0, CH // 2, unroll=2)
        def _(r):
            _rescale_row(rows_v.at[b], out_v.at[b], r, 0, rot_idx)
            _rescale_row(rows_v.at[b], out_v.at[b], r, 1, rot_idx)

        out_desc(t, b).start()

    prefetch(0, 0)
    prefetch(1, 1)
    process(0, 0, False)
    prefetch(2, 0)
    process(1, 1, False)
    prefetch(3, 1)

    def chunk_pair(i, carry):
        t = 2 * i
        process(t, 0, True)
        prefetch(t + 2, 0)
        process(t + 1, 1, True)
        prefetch(t + 3, 1)
        return carry

    lax.fori_loop(1, CHUNKS // 2 - 1, chunk_pair, 0)
    process(CHUNKS - 2, 0, True)
    process(CHUNKS - 1, 1, True)
    out_desc(0, 0).wait()
    out_desc(0, 1).wait()


def kernel(nouns_idx_tensor, vis_embeddings):
    idx2 = nouns_idx_tensor.reshape(R // IW, IW)
    out = _sc_lookup(idx2, vis_embeddings)
    return out.reshape(B, L, DIM)


# speedup vs baseline: 2.8967x; 1.0257x over previous
import functools

import jax
import jax.numpy as jnp
import numpy as np
from jax import lax
from jax.experimental import pallas as pl
from jax.experimental.pallas import tpu as pltpu
from jax.experimental.pallas import tpu_sc as plsc

NUM_EMB = 1000000
DIM = 64
B = 16384
L = 50
MAX_NORM = 3.0

NC = 2
NS = 16
NW = NC * NS
R = B * L
IW = 128
CH = 256
ROWS_PER_W = R // NW
CHUNKS = ROWS_PER_W // CH
STREAMS = CH // IW


def _rescale_row(rows_v, out_v, r, half, rot_idx):
    a = [rows_v[2 * r + half, pl.ds(q * 16, 16)] for q in range(DIM // 16)]
    sq = a[0] * a[0]
    for q in range(1, DIM // 16):
        sq = sq + a[q] * a[q]
    for rot in rot_idx:
        sq = sq + jnp.take_along_axis(sq, rot, axis=0)
    ii = plsc.bitcast(sq, jnp.int32)
    ii = 0x5F3759DF - (ii >> 1)
    y = plsc.bitcast(ii, jnp.float32)
    for _ in range(2):
        y = y * (1.5 - 0.5 * sq * y * y)
    scale = jnp.minimum(jnp.float32(1.0), jnp.float32(MAX_NORM) * y)
    for q in range(DIM // 16):
        out_v[r, pl.ds(half * DIM + q * 16, 16)] = a[q] * scale


@functools.partial(
    pl.kernel,
    out_type=jax.ShapeDtypeStruct((R // 2, 2 * DIM), jnp.float32),
    mesh=plsc.VectorSubcoreMesh(core_axis_name="c", subcore_axis_name="s"),
    compiler_params=pltpu.CompilerParams(
        needs_layout_passes=False, use_tc_tiling_on_sc=False
    ),
    scratch_types=[
        pltpu.VMEM((2, STREAMS, IW), jnp.int32),
        pltpu.VMEM((2, CH, DIM), jnp.float32),
        pltpu.VMEM((2, CH // 2, 2 * DIM), jnp.float32),
        pltpu.SemaphoreType.DMA,
        pltpu.SemaphoreType.DMA,
        pltpu.SemaphoreType.DMA,
        pltpu.SemaphoreType.DMA,
    ],
)
def _sc_lookup(
    idx_hbm, table_hbm, out_hbm, idx_v, rows_v, out_v, g0, g1, o0, o1
):
    wid = lax.axis_index("s") * NC + lax.axis_index("c")
    iota16 = lax.iota(jnp.int32, 16)
    rot_idx = [iota16 ^ (1 << j) for j in range(4)]
    gsem = (g0, g1)
    osem = (o0, o1)

    def gather_descs(t, b):
        return [
            pltpu.make_async_copy(
                table_hbm.at[idx_v.at[b, j]],
                rows_v.at[b, pl.ds(j * IW, IW)],
                gsem[b],
            )
            for j in range(STREAMS)
        ]

    def prefetch(t, b):
        r0 = wid * (ROWS_PER_W // IW) + t * STREAMS
        pltpu.sync_copy(idx_hbm.at[pl.ds(r0, STREAMS)], idx_v.at[b])
        for d in gather_descs(t, b):
            d.start()

    def out_desc(t, b):
        base2 = (wid * ROWS_PER_W + t * CH) // 2
        return pltpu.make_async_copy(
            out_v.at[b], out_hbm.at[pl.ds(base2, CH // 2)], osem[b]
        )

    def process(t, b, drain_out):
        for d in gather_descs(t, b):
            d.wait()
        if drain_out:
            out_desc(t, b).wait()

        @plsc.parallel_loop(0, CH // 2, unroll=4)
        def _(r):
            _rescale_row(rows_v.at[b], out_v.at[b], r, 0, rot_idx)
            _rescale_row(rows_v.at[b], out_v.at[b], r, 1, rot_idx)

        out_desc(t, b).start()

    prefetch(0, 0)
    prefetch(1, 1)
    process(0, 0, False)
    prefetch(2, 0)
    process(1, 1, False)
    prefetch(3, 1)

    def chunk_pair(i, carry):
        t = 2 * i
        process(t, 0, True)
        prefetch(t + 2, 0)
        process(t + 1, 1, True)
        prefetch(t + 3, 1)
        return carry

    lax.fori_loop(1, CHUNKS // 2 - 1, chunk_pair, 0)
    process(CHUNKS - 2, 0, True)
    process(CHUNKS - 1, 1, True)
    out_desc(0, 0).wait()
    out_desc(0, 1).wait()


def kernel(nouns_idx_tensor, vis_embeddings):
    idx2 = nouns_idx_tensor.reshape(R // IW, IW)
    out = _sc_lookup(idx2, vis_embeddings)
    return out.reshape(B, L, DIM)
